# baseline (speedup 1.0000x reference)
import functools

import jax
import jax.numpy as jnp
from jax.experimental import pallas as pl
from jax.experimental.pallas import tpu as pltpu

_EPS = 1e-5
_K = 20


def _lrelu(z):
    return jnp.maximum(z, 0.2 * z)


def _knn_feat_kernel(xt_ref, xr_ref, wa_ref, wc_ref, we_ref, b2_ref,
                     hmax_ref, hsum_ref, p_ref):
    N = xt_ref.shape[1]
    xt = xt_ref[0]
    xr = xr_ref[0]
    p = 2.0 * jnp.dot(xt, xr, preferred_element_type=jnp.float32)
    xxr = jnp.sum(xr * xr, axis=0, keepdims=True)
    p = p - xxr

    ya = jnp.dot(xt, wa_ref[...], preferred_element_type=jnp.float32)
    yc = jnp.dot(xt, wc_ref[...], preferred_element_type=jnp.float32)
    ya1 = jnp.concatenate([ya, jnp.ones((N, 1), jnp.float32)],
                          axis=1).astype(jnp.bfloat16)

    neg = jnp.float32(-1e38)
    rows = jax.lax.broadcasted_iota(jnp.int32, (N, N), 0)
    cols = jax.lax.broadcasted_iota(jnp.int32, (N, N), 1)
    p_ref[...] = jnp.where(rows == cols, neg, p)

    def body(_, acc):
        p = p_ref[...]
        m = jnp.max(p, axis=1, keepdims=True)
        sel = p == m
        f = sel.astype(jnp.bfloat16)
        g1 = jnp.dot(f, ya1, preferred_element_type=jnp.float32)
        g = g1[:, :64]
        cnt = g1[:, 64:65]
        rfac = jnp.where(cnt > 1.5, 1.0 / jnp.maximum(cnt, 1.0),
                         jnp.float32(1.0))
        acc = jnp.maximum(acc, g * rfac)
        p_ref[...] = jnp.where(sel, neg, p)
        return acc

    acc = jax.lax.fori_loop(0, _K - 1, body, ya)

    x1 = _lrelu(acc + yc)
    he = jnp.dot(x1, we_ref[...], preferred_element_type=jnp.float32)
    he = _lrelu(he + b2_ref[...])
    hmax_ref[0] = jnp.max(he, axis=0, keepdims=True)
    hsum_ref[0] = jnp.sum(he, axis=0, keepdims=True)


def _tail_kernel(hmax_ref, hsum_ref, awgn_ref,
                 w2_ref, bb2_ref, w3_ref, bb3_ref, w4_ref, bb4_ref,
                 w5_ref, bb5_ref, w6_ref, bb6_ref, out_ref, *, n_points):
    h = jnp.concatenate(
        [hmax_ref[...], hsum_ref[...] * (1.0 / n_points)], axis=1)
    h = _lrelu(jnp.dot(h, w2_ref[...], preferred_element_type=jnp.float32)
               + bb2_ref[...])
    nrm = jnp.sqrt(jnp.sum(h * h, axis=1, keepdims=True))
    h = h / jnp.maximum(nrm, 1e-12) + 0.1 * awgn_ref[...]
    for w, bb in ((w3_ref, bb3_ref), (w4_ref, bb4_ref),
                  (w5_ref, bb5_ref), (w6_ref, bb6_ref)):
        h = _lrelu(jnp.dot(h, w[...], preferred_element_type=jnp.float32)
                   + bb[...])
    out_ref[...] = h


def _fold_bn(bn):
    g, b, m, v = bn
    s = g / jnp.sqrt(v + _EPS)
    return s, b - m * s


def kernel(x, params, awgn_noise):
    B, C, N = x.shape
    f32 = jnp.float32

    x8 = jnp.concatenate(
        [x, jnp.ones((B, 1, N), f32), jnp.zeros((B, 4, N), f32)], axis=1)
    xt8 = jnp.transpose(x8, (0, 2, 1))

    s1, bb1 = _fold_bn(params['bn_conv1'])
    W1 = params['W_conv1']
    Wa = W1[:, :3] * s1[:, None]
    Wc = (W1[:, 3:] - W1[:, :3]) * s1[:, None]
    WaT = jnp.zeros((8, 64), f32).at[:3].set(Wa.T)
    WcT = jnp.zeros((8, 64), f32).at[:3].set(Wc.T).at[3].set(bb1)

    s2, bb2 = _fold_bn(params['bn_exit_conv'])
    WeT = (params['W_exit_conv'] * s2[:, None]).T
    bb2r = bb2.reshape(1, 256)

    grid = (B,)
    hmax, hsum = pl.pallas_call(
        _knn_feat_kernel,
        grid=grid,
        in_specs=[
            pl.BlockSpec((1, N, 8), lambda b: (b, 0, 0)),
            pl.BlockSpec((1, 8, N), lambda b: (b, 0, 0)),
            pl.BlockSpec((8, 64), lambda b: (0, 0)),
            pl.BlockSpec((8, 64), lambda b: (0, 0)),
            pl.BlockSpec((64, 256), lambda b: (0, 0)),
            pl.BlockSpec((1, 256), lambda b: (0, 0)),
        ],
        out_specs=[
            pl.BlockSpec((1, 1, 256), lambda b: (b, 0, 0)),
            pl.BlockSpec((1, 1, 256), lambda b: (b, 0, 0)),
        ],
        out_shape=[
            jax.ShapeDtypeStruct((B, 1, 256), f32),
            jax.ShapeDtypeStruct((B, 1, 256), f32),
        ],
        scratch_shapes=[pltpu.VMEM((N, N), f32)],
        compiler_params=pltpu.CompilerParams(
            dimension_semantics=("parallel",)),
    )(xt8, x8, WaT, WcT, WeT, bb2r)

    hmax = hmax.reshape(B, 256)
    hsum = hsum.reshape(B, 256)

    def fold_linear(Wkey, bkey, bnkey, pad_to=None):
        s, t = _fold_bn(params[bnkey])
        W = params[Wkey] * s[:, None]
        bb = params[bkey] * s + t
        Wt = W.T
        if pad_to is not None and Wt.shape[1] < pad_to:
            Wt = jnp.pad(Wt, ((0, 0), (0, pad_to - Wt.shape[1])))
            bb = jnp.pad(bb, (0, pad_to - bb.shape[0]))
        return Wt, bb.reshape(1, -1)

    W2, bv2 = fold_linear('W_fc2', 'b_fc2', 'bn_fc2')
    W3, bv3 = fold_linear('W_p1', 'b_p1', 'bn_p1')
    W4, bv4 = fold_linear('W_p2', 'b_p2', 'bn_p2')
    W5, bv5 = fold_linear('W_p3', 'b_p3', 'bn_p3')
    W6, bv6 = fold_linear('W_p4', 'b_p4', 'bn_p4', pad_to=128)

    full = lambda shape: pl.BlockSpec(shape, lambda: (0,) * len(shape))
    out = pl.pallas_call(
        functools.partial(_tail_kernel, n_points=N),
        in_specs=[
            full((B, 256)), full((B, 256)), full((B, 1536)),
            full((512, 1536)), full((1, 1536)),
            full((1536, 512)), full((1, 512)),
            full((512, 256)), full((1, 256)),
            full((256, 128)), full((1, 128)),
            full((128, 128)), full((1, 128)),
        ],
        out_specs=full((B, 128)),
        out_shape=jax.ShapeDtypeStruct((B, 128), f32),
    )(hmax, hsum, awgn_noise,
      W2, bv2, W3, bv3, W4, bv4, W5, bv5, W6, bv6)

    return out[:, :40]


# speedup vs baseline: 1.0255x; 1.0255x over previous
import functools

import jax
import jax.numpy as jnp
from jax.experimental import pallas as pl
from jax.experimental.pallas import tpu as pltpu

_EPS = 1e-5
_K = 20


def _lrelu(z):
    return jnp.maximum(z, 0.2 * z)


def _knn_feat_kernel(xt_ref, xr_ref, wa_ref, wc_ref, we_ref, b2_ref,
                     hmax_ref, hsum_ref, p_ref):
    N = xt_ref.shape[1]
    xt = xt_ref[0]
    xr = xr_ref[0]
    p = 2.0 * jnp.dot(xt, xr, preferred_element_type=jnp.float32)
    xxr = jnp.sum(xr * xr, axis=0, keepdims=True)
    p = p - xxr

    ya = jnp.dot(xt, wa_ref[...], preferred_element_type=jnp.float32)
    yc = jnp.dot(xt, wc_ref[...], preferred_element_type=jnp.float32)
    ya1 = jnp.concatenate([ya, jnp.ones((N, 1), jnp.float32)], axis=1)

    neg = jnp.float32(-1e38)
    rows = jax.lax.broadcasted_iota(jnp.int32, (N, N), 0)
    cols = jax.lax.broadcasted_iota(jnp.int32, (N, N), 1)
    p_ref[...] = jnp.where(rows == cols, neg, p)

    def body(_, acc):
        p = p_ref[...]
        m = jnp.max(p, axis=1, keepdims=True)
        sel = p == m
        f = sel.astype(jnp.float32)
        g1 = jnp.dot(f, ya1, preferred_element_type=jnp.float32)
        g = g1[:, :64]
        cnt = g1[:, 64:65]
        rfac = jnp.where(cnt > 1.5, 1.0 / jnp.maximum(cnt, 1.0),
                         jnp.float32(1.0))
        acc = jnp.maximum(acc, g * rfac)
        p_ref[...] = jnp.where(sel, neg, p)
        return acc

    acc = jax.lax.fori_loop(0, _K - 1, body, ya)

    x1 = _lrelu(acc + yc)
    he = jnp.dot(x1, we_ref[...], preferred_element_type=jnp.float32)
    he = _lrelu(he + b2_ref[...])
    hmax_ref[0] = jnp.max(he, axis=0, keepdims=True)
    hsum_ref[0] = jnp.sum(he, axis=0, keepdims=True)


def _tail_kernel(hmax_ref, hsum_ref, awgn_ref,
                 w2_ref, bb2_ref, w3_ref, bb3_ref, w4_ref, bb4_ref,
                 w5_ref, bb5_ref, w6_ref, bb6_ref, out_ref, *, n_points):
    h = jnp.concatenate(
        [hmax_ref[...], hsum_ref[...] * (1.0 / n_points)], axis=1)
    h = _lrelu(jnp.dot(h, w2_ref[...], preferred_element_type=jnp.float32)
               + bb2_ref[...])
    nrm = jnp.sqrt(jnp.sum(h * h, axis=1, keepdims=True))
    h = h / jnp.maximum(nrm, 1e-12) + 0.1 * awgn_ref[...]
    for w, bb in ((w3_ref, bb3_ref), (w4_ref, bb4_ref),
                  (w5_ref, bb5_ref), (w6_ref, bb6_ref)):
        h = _lrelu(jnp.dot(h, w[...], preferred_element_type=jnp.float32)
                   + bb[...])
    out_ref[...] = h


def _fold_bn(bn):
    g, b, m, v = bn
    s = g / jnp.sqrt(v + _EPS)
    return s, b - m * s


def kernel(x, params, awgn_noise):
    B, C, N = x.shape
    f32 = jnp.float32

    x8 = jnp.concatenate(
        [x, jnp.ones((B, 1, N), f32), jnp.zeros((B, 4, N), f32)], axis=1)
    xt8 = jnp.transpose(x8, (0, 2, 1))

    s1, bb1 = _fold_bn(params['bn_conv1'])
    W1 = params['W_conv1']
    Wa = W1[:, :3] * s1[:, None]
    Wc = (W1[:, 3:] - W1[:, :3]) * s1[:, None]
    WaT = jnp.zeros((8, 64), f32).at[:3].set(Wa.T)
    WcT = jnp.zeros((8, 64), f32).at[:3].set(Wc.T).at[3].set(bb1)

    s2, bb2 = _fold_bn(params['bn_exit_conv'])
    WeT = (params['W_exit_conv'] * s2[:, None]).T
    bb2r = bb2.reshape(1, 256)

    grid = (B,)
    hmax, hsum = pl.pallas_call(
        _knn_feat_kernel,
        grid=grid,
        in_specs=[
            pl.BlockSpec((1, N, 8), lambda b: (b, 0, 0)),
            pl.BlockSpec((1, 8, N), lambda b: (b, 0, 0)),
            pl.BlockSpec((8, 64), lambda b: (0, 0)),
            pl.BlockSpec((8, 64), lambda b: (0, 0)),
            pl.BlockSpec((64, 256), lambda b: (0, 0)),
            pl.BlockSpec((1, 256), lambda b: (0, 0)),
        ],
        out_specs=[
            pl.BlockSpec((1, 1, 256), lambda b: (b, 0, 0)),
            pl.BlockSpec((1, 1, 256), lambda b: (b, 0, 0)),
        ],
        out_shape=[
            jax.ShapeDtypeStruct((B, 1, 256), f32),
            jax.ShapeDtypeStruct((B, 1, 256), f32),
        ],
        scratch_shapes=[pltpu.VMEM((N, N), f32)],
        compiler_params=pltpu.CompilerParams(
            dimension_semantics=("parallel",)),
    )(xt8, x8, WaT, WcT, WeT, bb2r)

    hmax = hmax.reshape(B, 256)
    hsum = hsum.reshape(B, 256)

    def fold_linear(Wkey, bkey, bnkey, pad_to=None):
        s, t = _fold_bn(params[bnkey])
        W = params[Wkey] * s[:, None]
        bb = params[bkey] * s + t
        Wt = W.T
        if pad_to is not None and Wt.shape[1] < pad_to:
            Wt = jnp.pad(Wt, ((0, 0), (0, pad_to - Wt.shape[1])))
            bb = jnp.pad(bb, (0, pad_to - bb.shape[0]))
        return Wt, bb.reshape(1, -1)

    W2, bv2 = fold_linear('W_fc2', 'b_fc2', 'bn_fc2')
    W3, bv3 = fold_linear('W_p1', 'b_p1', 'bn_p1')
    W4, bv4 = fold_linear('W_p2', 'b_p2', 'bn_p2')
    W5, bv5 = fold_linear('W_p3', 'b_p3', 'bn_p3')
    W6, bv6 = fold_linear('W_p4', 'b_p4', 'bn_p4', pad_to=128)

    full = lambda shape: pl.BlockSpec(shape, lambda: (0,) * len(shape))
    out = pl.pallas_call(
        functools.partial(_tail_kernel, n_points=N),
        in_specs=[
            full((B, 256)), full((B, 256)), full((B, 1536)),
            full((512, 1536)), full((1, 1536)),
            full((1536, 512)), full((1, 512)),
            full((512, 256)), full((1, 256)),
            full((256, 128)), full((1, 128)),
            full((128, 128)), full((1, 128)),
        ],
        out_specs=full((B, 128)),
        out_shape=jax.ShapeDtypeStruct((B, 128), f32),
    )(hmax, hsum, awgn_noise,
      W2, bv2, W3, bv3, W4, bv4, W5, bv5, W6, bv6)

    return out[:, :40]
